# baseline (speedup 1.0000x reference)
import jax
import jax.numpy as jnp
from jax.experimental import pallas as pl

H, W = 50, 37
HP, WP = H + 2, W + 2
NFLAT = HP * WP
NROWS = 2032
NC = 1948
CIN = 512
COUT = 512


def _rpn_kernel(x_ref, wt_ref, bc_ref, wcl_ref, bcl_ref, cls_ref, loc_ref):
    acc = jnp.zeros((NC, COUT), jnp.float32)
    for t in range(9):
        dy, dx = t // 3, t % 3
        off = dy * WP + dx
        acc = acc + jnp.dot(x_ref[off:off + NC, :], wt_ref[t * CIN:(t + 1) * CIN, :],
                            preferred_element_type=jnp.float32)
    h = jnp.maximum(acc + bc_ref[:1, :], 0.0)
    s_t = jnp.dot(h, wcl_ref[...],
                  preferred_element_type=jnp.float32) + bcl_ref[:1, :]
    s = jnp.transpose(s_t)
    a = s[0:9]
    b = s[9:18]
    m = jnp.maximum(a, b)
    ea = jnp.exp(a - m)
    eb = jnp.exp(b - m)
    d = ea + eb
    cls_ref[0, :, :NC] = jnp.concatenate([ea / d, eb / d], axis=0)
    loc_ref[0, :, :NC] = s[18:54]


def kernel(feats, gt_boxes, im_info, W_conv, b_conv, W_cls, b_cls, W_loc, b_loc):
    B = feats.shape[0]
    xp = jnp.pad(feats, ((0, 0), (0, 0), (1, 1), (1, 1)))
    xt = jnp.transpose(xp.reshape(B, CIN, NFLAT), (0, 2, 1)).astype(jnp.bfloat16)
    xt = jnp.pad(xt, ((0, 0), (0, NROWS - NFLAT), (0, 0))).reshape(B * NROWS, CIN)
    wbf = jax.lax.optimization_barrier(W_conv.astype(jnp.bfloat16))
    wt = jnp.transpose(wbf, (2, 3, 1, 0)).reshape(9 * CIN, COUT)
    wcl = jnp.concatenate([W_cls[:, :, 0, 0], W_loc[:, :, 0, 0]], axis=0)
    wclt = jnp.transpose(wcl)
    bcl = jnp.concatenate([b_cls, b_loc])[None, :]
    bc = b_conv[None, :]

    cls_flat, loc_flat = pl.pallas_call(
        _rpn_kernel,
        grid=(B,),
        in_specs=[
            pl.BlockSpec((NROWS, CIN), lambda i: (i, 0)),
            pl.BlockSpec((9 * CIN, COUT), lambda i: (0, 0)),
            pl.BlockSpec((1, COUT), lambda i: (0, 0)),
            pl.BlockSpec((CIN, 54), lambda i: (0, 0)),
            pl.BlockSpec((1, 54), lambda i: (0, 0)),
        ],
        out_specs=[
            pl.BlockSpec((1, 18, H * WP), lambda i: (i, 0, 0)),
            pl.BlockSpec((1, 36, H * WP), lambda i: (i, 0, 0)),
        ],
        out_shape=[
            jax.ShapeDtypeStruct((B, 18, H * WP), jnp.float32),
            jax.ShapeDtypeStruct((B, 36, H * WP), jnp.float32),
        ],
    )(xt, wt, bc, wclt, bcl)

    cls = cls_flat.reshape(B, 18, H, WP)[:, :, :, :W]
    loc = loc_flat.reshape(B, 36, H, WP)[:, :, :, :W]
    return (cls, loc)


# speedup vs baseline: 1.0007x; 1.0007x over previous
import jax
import jax.numpy as jnp
from jax.experimental import pallas as pl

H, W = 50, 37
HP, WP = H + 2, W + 2
NFLAT = HP * WP
NROWS = 2032
NC = 1948
CIN = 512
COUT = 512


def _rpn_kernel(x_ref, wt_ref, bc_ref, wcl_ref, bcl_ref, cls_ref, loc_ref):
    xcat = jnp.concatenate(
        [x_ref[(t // 3) * WP + (t % 3):(t // 3) * WP + (t % 3) + NC, :]
         for t in range(9)], axis=1)
    acc = jnp.dot(xcat, wt_ref[...], preferred_element_type=jnp.float32)
    h = jnp.maximum(acc + bc_ref[:1, :], 0.0)
    s_t = jnp.dot(h, wcl_ref[...],
                  preferred_element_type=jnp.float32) + bcl_ref[:1, :]
    s = jnp.transpose(s_t)
    a = s[0:9]
    b = s[9:18]
    m = jnp.maximum(a, b)
    ea = jnp.exp(a - m)
    eb = jnp.exp(b - m)
    d = ea + eb
    cls_ref[0, :, :NC] = jnp.concatenate([ea / d, eb / d], axis=0)
    loc_ref[0, :, :NC] = s[18:54]


def kernel(feats, gt_boxes, im_info, W_conv, b_conv, W_cls, b_cls, W_loc, b_loc):
    B = feats.shape[0]
    xp = jnp.pad(feats, ((0, 0), (0, 0), (1, 1), (1, 1)))
    xf = jnp.pad(xp.reshape(B, CIN, NFLAT), ((0, 0), (0, 0), (0, NROWS - NFLAT)))
    xt = jnp.transpose(xf, (0, 2, 1)).astype(jnp.bfloat16).reshape(B * NROWS, CIN)
    wbf = jax.lax.optimization_barrier(W_conv.astype(jnp.bfloat16))
    wt = jnp.transpose(wbf, (2, 3, 1, 0)).reshape(9 * CIN, COUT)
    wcl = jnp.concatenate([W_cls[:, :, 0, 0], W_loc[:, :, 0, 0]], axis=0)
    wclt = jnp.transpose(wcl)
    bcl = jnp.concatenate([b_cls, b_loc])[None, :]
    bc = b_conv[None, :]

    cls_flat, loc_flat = pl.pallas_call(
        _rpn_kernel,
        grid=(B,),
        in_specs=[
            pl.BlockSpec((NROWS, CIN), lambda i: (i, 0)),
            pl.BlockSpec((9 * CIN, COUT), lambda i: (0, 0)),
            pl.BlockSpec((1, COUT), lambda i: (0, 0)),
            pl.BlockSpec((CIN, 54), lambda i: (0, 0)),
            pl.BlockSpec((1, 54), lambda i: (0, 0)),
        ],
        out_specs=[
            pl.BlockSpec((1, 18, H * WP), lambda i: (i, 0, 0)),
            pl.BlockSpec((1, 36, H * WP), lambda i: (i, 0, 0)),
        ],
        out_shape=[
            jax.ShapeDtypeStruct((B, 18, H * WP), jnp.float32),
            jax.ShapeDtypeStruct((B, 36, H * WP), jnp.float32),
        ],
    )(xt, wt, bc, wclt, bcl)

    cls = cls_flat.reshape(B, 18, H, WP)[:, :, :, :W]
    loc = loc_flat.reshape(B, 36, H, WP)[:, :, :, :W]
    return (cls, loc)


# speedup vs baseline: 1.0825x; 1.0818x over previous
import jax
import jax.numpy as jnp
from jax.experimental import pallas as pl

H, W = 50, 37
HP, WP = H + 2, W + 2
NFLAT = HP * WP
NROWS = 2032
NC = 1948
CIN = 512
COUT = 512


def _rpn_kernel(x_ref, wt_ref, bc_ref, wcl_ref, bcl_ref, cls_ref, loc_ref):
    xcat = jnp.concatenate(
        [x_ref[0, (t // 3) * WP + (t % 3):(t // 3) * WP + (t % 3) + NC, :]
         for t in range(9)], axis=1)
    acc = jnp.dot(xcat, wt_ref[...], preferred_element_type=jnp.float32)
    h = jnp.maximum(acc + bc_ref[:1, :], 0.0)
    s_t = jnp.dot(h, wcl_ref[...],
                  preferred_element_type=jnp.float32) + bcl_ref[:1, :]
    s = jnp.transpose(s_t)
    a = s[0:9]
    b = s[9:18]
    m = jnp.maximum(a, b)
    ea = jnp.exp(a - m)
    eb = jnp.exp(b - m)
    d = ea + eb
    cls_ref[0, :, :NC] = jnp.concatenate([ea / d, eb / d], axis=0)
    loc_ref[0, :, :NC] = s[18:54]


def kernel(feats, gt_boxes, im_info, W_conv, b_conv, W_cls, b_cls, W_loc, b_loc):
    B = feats.shape[0]
    xp = jnp.pad(feats, ((0, 0), (0, 0), (1, 1), (1, 1)))
    xt = jnp.transpose(xp.reshape(B, CIN, NFLAT), (0, 2, 1)).astype(jnp.bfloat16)
    wbf = jax.lax.optimization_barrier(W_conv.astype(jnp.bfloat16))
    wt = jnp.transpose(wbf, (2, 3, 1, 0)).reshape(9 * CIN, COUT)
    wcl = jnp.concatenate([W_cls[:, :, 0, 0], W_loc[:, :, 0, 0]], axis=0)
    wclt = jnp.transpose(wcl)
    bcl = jnp.concatenate([b_cls, b_loc])[None, :]
    bc = b_conv[None, :]

    cls_flat, loc_flat = pl.pallas_call(
        _rpn_kernel,
        grid=(B,),
        in_specs=[
            pl.BlockSpec((1, NFLAT, CIN), lambda i: (i, 0, 0)),
            pl.BlockSpec((9 * CIN, COUT), lambda i: (0, 0)),
            pl.BlockSpec((1, COUT), lambda i: (0, 0)),
            pl.BlockSpec((CIN, 54), lambda i: (0, 0)),
            pl.BlockSpec((1, 54), lambda i: (0, 0)),
        ],
        out_specs=[
            pl.BlockSpec((1, 18, H * WP), lambda i: (i, 0, 0)),
            pl.BlockSpec((1, 36, H * WP), lambda i: (i, 0, 0)),
        ],
        out_shape=[
            jax.ShapeDtypeStruct((B, 18, H * WP), jnp.float32),
            jax.ShapeDtypeStruct((B, 36, H * WP), jnp.float32),
        ],
    )(xt, wt, bc, wclt, bcl)

    cls = cls_flat.reshape(B, 18, H, WP)[:, :, :, :W]
    loc = loc_flat.reshape(B, 36, H, WP)[:, :, :, :W]
    return (cls, loc)
